# baseline (speedup 1.0000x reference)
import jax
import jax.numpy as jnp
from jax.experimental import pallas as pl
from jax.experimental.pallas import tpu as pltpu

_BLKL = 2048
_NEG_INF = float("-inf")
_IBIG = jnp.iinfo(jnp.int32).max


def _assign_kernel(anchT_ref, ann_ref, bto_ref, bti_ref, bpi_ref,
                   cmax_ref, carg_ref):
    i = pl.program_id(1)
    nblk = pl.num_programs(1)
    at = anchT_ref[...]
    ann = ann_ref[0]
    G = ann.shape[0]
    a0, a1, a2, a3 = at[0:1, :], at[1:2, :], at[2:3, :], at[3:4, :]
    b0, b1 = ann[:, 0:1], ann[:, 1:2]
    b2, b3 = ann[:, 2:3], ann[:, 3:4]
    valid = ann[:, 4:5] != -1.0
    area_b = (b2 - b0) * (b3 - b1)
    iw = jnp.maximum(jnp.minimum(a2, b2) - jnp.maximum(a0, b0), 0.0)
    ih = jnp.maximum(jnp.minimum(a3, b3) - jnp.maximum(a1, b1), 0.0)
    inter = iw * ih
    ua = jnp.maximum((a2 - a0) * (a3 - a1) + area_b - inter, 1e-8)
    iou = jnp.where(valid, inter / ua, _NEG_INF)

    sidx = jax.lax.broadcasted_iota(jnp.int32, (G, 1), 0)
    bto = jnp.max(iou, axis=0, keepdims=True)
    bti = jnp.min(jnp.where(iou == bto, sidx, _IBIG), axis=0,
                  keepdims=True)
    bto_ref[0] = bto
    bti_ref[0] = bti

    m = jnp.max(iou, axis=1, keepdims=True)
    lidx = jax.lax.broadcasted_iota(jnp.int32, (1, iou.shape[1]), 1)
    a = jnp.min(jnp.where(iou == m, lidx, _IBIG), axis=1,
                keepdims=True) + i * _BLKL

    @pl.when(i == 0)
    def _():
        cmax_ref[...] = m
        carg_ref[...] = a

    @pl.when(i > 0)
    def _():
        upd = m > cmax_ref[...]
        cmax_ref[...] = jnp.where(upd, m, cmax_ref[...])
        carg_ref[...] = jnp.where(upd, a, carg_ref[...])

    @pl.when(i == nblk - 1)
    def _():
        bpi_ref[0] = carg_ref[...]


def _make_loss_kernel(A):
    def _loss_kernel(cls_ref, regT_ref, anchT_ref, ann1_ref, ann2_ref,
                     annT1_ref, annT2_ref, bto_ref, bti_ref, bpi_ref,
                     out_ref):
        i = pl.program_id(1)
        C = cls_ref.shape[2]
        ann = ann1_ref[0]
        annn = ann2_ref[0]
        annT = annT1_ref[0]
        annnT = annT2_ref[0]
        bpi = bpi_ref[0]
        G = ann.shape[0]
        blkl = bto_ref.shape[2]
        valid = ann[:, 4:5] != -1.0
        validn = annn[:, 4:5] != -1.0
        sidx = jax.lax.broadcasted_iota(jnp.int32, (G, 1), 0)
        lidx = jax.lax.broadcasted_iota(jnp.int32, (1, blkl), 1)
        gidl = i * blkl + lidx
        in_range = gidl < A

        bto = bto_ref[0]
        bti = bti_ref[0]

        is_best_for = (bpi == gidl) & valid
        chosen = jnp.max(jnp.where(is_best_for, sidx, -1), axis=0,
                         keepdims=True)
        is_best = chosen >= 0
        bto = jnp.where(is_best, 2.0, bto)
        bti = jnp.where(is_best, chosen, bti)

        pos = bto >= 0.5
        neg = bto < 0.4
        active = (pos | neg) & in_range
        npos = jnp.sum(jnp.where(pos, 1.0, 0.0))

        onehot = jnp.where(sidx == bti, 1.0, 0.0)
        asn6 = jax.lax.dot_general(annT, onehot, (((1,), (0,)), ((), ())),
                                   preferred_element_type=jnp.float32)
        labels = asn6[4:5, :].astype(jnp.int32)
        lab_ok = (labels >= 0) & (labels < C)

        eq = (annn[:, 5:6] == asn6[5:6, :]) & validn
        has_match = jnp.max(jnp.where(eq, 1.0, 0.0), axis=0,
                            keepdims=True) == 1.0
        fm = jnp.min(jnp.where(eq, sidx, _IBIG), axis=0, keepdims=True)
        nxt_oh = jnp.where(sidx == fm, 1.0, 0.0)
        ank6 = jax.lax.dot_general(annnT, nxt_oh, (((1,), (0,)), ((), ())),
                                   preferred_element_type=jnp.float32)
        hm_f = jnp.where(has_match, 1.0, 0.0)
        ank = [ank6[k:k + 1, :] * hm_f for k in range(4)]

        labels_s = jnp.transpose(labels)
        c = jnp.clip(cls_ref[0], 1e-4, 1.0 - 1e-4)
        l0 = 0.75 * (c * c) * (-jnp.log(1.0 - c))
        s0 = jnp.sum(l0, axis=1, keepdims=True)
        clsi = jax.lax.broadcasted_iota(jnp.int32, (1, C), 1)
        sel = clsi == labels_s
        c_lab = jnp.sum(jnp.where(sel, c, 0.0), axis=1, keepdims=True)
        s0_l = jnp.transpose(s0)
        c_lab_l = jnp.transpose(c_lab)
        one_m = 1.0 - c_lab_l
        corr = 0.25 * (one_m * one_m) * (-jnp.log(c_lab_l)) \
            - 0.75 * (c_lab_l * c_lab_l) * (-jnp.log(one_m))
        cls_sum = jnp.sum(jnp.where(active, s0_l, 0.0)) \
            + jnp.sum(jnp.where(pos & lab_ok, corr, 0.0))

        at = anchT_ref[...]
        a0, a1, a2, a3 = at[0:1, :], at[1:2, :], at[2:3, :], at[3:4, :]
        aw = a2 - a0
        ah = a3 - a1
        acx = a0 + 0.5 * aw
        acy = a1 + 0.5 * ah
        gw_raw = asn6[2:3, :] - asn6[0:1, :]
        gh_raw = asn6[3:4, :] - asn6[1:2, :]
        gcx = asn6[0:1, :] + 0.5 * gw_raw
        gcy = asn6[1:2, :] + 0.5 * gh_raw
        gwn_raw = ank[2] - ank[0]
        ghn_raw = ank[3] - ank[1]
        gcxn = ank[0] + 0.5 * gwn_raw
        gcyn = ank[1] + 0.5 * ghn_raw
        gw = jnp.maximum(gw_raw, 1.0)
        gh = jnp.maximum(gh_raw, 1.0)
        gwn = jnp.maximum(gwn_raw, 1.0)
        ghn = jnp.maximum(ghn_raw, 1.0)

        t_cols = (
            (gcx - acx) / aw / 0.1,
            (gcy - acy) / ah / 0.1,
            jnp.log(gw / aw) / 0.2,
            jnp.log(gh / ah) / 0.2,
            (gcxn - acx) / aw / 0.1,
            (gcyn - acy) / ah / 0.1,
            jnp.log(gwn / aw) / 0.2,
            jnp.log(ghn / ah) / 0.2,
        )
        regT = jnp.transpose(regT_ref[0])
        reg_sum = 0.0
        for k in range(8):
            rd = jnp.abs(t_cols[k] - regT[k:k + 1, :])
            if k >= 4:
                rd = rd * hm_f
            rl = jnp.where(rd <= 1.0 / 9.0, 0.5 * 9.0 * rd * rd,
                           rd - 0.5 / 9.0)
            reg_sum = reg_sum + jnp.sum(jnp.where(pos, rl, 0.0))

        lane8 = jax.lax.broadcasted_iota(jnp.int32, (1, 8), 1)
        vec = jnp.where(lane8 == 0, cls_sum,
                        jnp.where(lane8 == 1, reg_sum,
                                  jnp.where(lane8 == 2, npos, 0.0)))

        @pl.when(i == 0)
        def _():
            out_ref[0] = vec

        @pl.when(i > 0)
        def _():
            out_ref[0] = out_ref[0] + vec

    return _loss_kernel


def kernel(classifications, regressions, anchors, annotations1, annotations2):
    B, A, C = classifications.shape
    G = annotations1.shape[1]
    a_pad = (-A) % _BLKL
    a_tot = A + a_pad
    nblk = a_tot // _BLKL

    anchT = jnp.pad(anchors[0].astype(jnp.float32).T, ((0, 0), (0, a_pad)),
                    constant_values=-1e30)
    ann1 = annotations1.astype(jnp.float32)
    ann2 = annotations2.astype(jnp.float32)
    annT1 = ann1.transpose(0, 2, 1)
    annT2 = ann2.transpose(0, 2, 1)

    bto_all, bti_all, bpi = pl.pallas_call(
        _assign_kernel,
        grid=(B, nblk),
        in_specs=[
            pl.BlockSpec((4, _BLKL), lambda b, i: (0, i)),
            pl.BlockSpec((1, G, 6), lambda b, i: (b, 0, 0)),
        ],
        out_specs=[
            pl.BlockSpec((1, 1, _BLKL), lambda b, i: (b, 0, i)),
            pl.BlockSpec((1, 1, _BLKL), lambda b, i: (b, 0, i)),
            pl.BlockSpec((1, G, 1), lambda b, i: (b, 0, 0)),
        ],
        out_shape=[
            jax.ShapeDtypeStruct((B, 1, a_tot), jnp.float32),
            jax.ShapeDtypeStruct((B, 1, a_tot), jnp.int32),
            jax.ShapeDtypeStruct((B, G, 1), jnp.int32),
        ],
        scratch_shapes=[
            pltpu.VMEM((G, 1), jnp.float32),
            pltpu.VMEM((G, 1), jnp.int32),
        ],
    )(anchT, ann1)

    sums = pl.pallas_call(
        _make_loss_kernel(A),
        grid=(B, nblk),
        in_specs=[
            pl.BlockSpec((1, _BLKL, C), lambda b, i: (b, i, 0)),
            pl.BlockSpec((1, _BLKL, 8), lambda b, i: (b, i, 0)),
            pl.BlockSpec((4, _BLKL), lambda b, i: (0, i)),
            pl.BlockSpec((1, G, 6), lambda b, i: (b, 0, 0)),
            pl.BlockSpec((1, G, 6), lambda b, i: (b, 0, 0)),
            pl.BlockSpec((1, 6, G), lambda b, i: (b, 0, 0)),
            pl.BlockSpec((1, 6, G), lambda b, i: (b, 0, 0)),
            pl.BlockSpec((1, 1, _BLKL), lambda b, i: (b, 0, i)),
            pl.BlockSpec((1, 1, _BLKL), lambda b, i: (b, 0, i)),
            pl.BlockSpec((1, G, 1), lambda b, i: (b, 0, 0)),
        ],
        out_specs=pl.BlockSpec((1, 1, 8), lambda b, i: (b, 0, 0)),
        out_shape=jax.ShapeDtypeStruct((B, 1, 8), jnp.float32),
    )(classifications, regressions, anchT, ann1, ann2, annT1, annT2,
      bto_all, bti_all, bpi)

    cls_s = sums[:, 0, 0]
    reg_s = sums[:, 0, 1]
    npos = sums[:, 0, 2]
    cls_losses = cls_s / jnp.maximum(npos, 1.0)
    reg_losses = reg_s / jnp.maximum(npos * 8.0, 1.0)
    return (jnp.mean(cls_losses, keepdims=True),
            jnp.mean(reg_losses, keepdims=True))


# speedup vs baseline: 1.0881x; 1.0881x over previous
import jax
import jax.numpy as jnp
from jax.experimental import pallas as pl
from jax.experimental.pallas import tpu as pltpu

_BLKL = 4096
_NEG_INF = float("-inf")
_IBIG = jnp.iinfo(jnp.int32).max


def _assign_kernel(anchT_ref, ann_ref, bto_ref, bti_ref, bpi_ref,
                   cmax_ref, carg_ref):
    i = pl.program_id(1)
    nblk = pl.num_programs(1)
    at = anchT_ref[...]
    ann = ann_ref[0]
    G = ann.shape[0]
    a0, a1, a2, a3 = at[0:1, :], at[1:2, :], at[2:3, :], at[3:4, :]
    b0, b1 = ann[:, 0:1], ann[:, 1:2]
    b2, b3 = ann[:, 2:3], ann[:, 3:4]
    valid = ann[:, 4:5] != -1.0
    area_b = (b2 - b0) * (b3 - b1)
    iw = jnp.maximum(jnp.minimum(a2, b2) - jnp.maximum(a0, b0), 0.0)
    ih = jnp.maximum(jnp.minimum(a3, b3) - jnp.maximum(a1, b1), 0.0)
    inter = iw * ih
    ua = jnp.maximum((a2 - a0) * (a3 - a1) + area_b - inter, 1e-8)
    iou = jnp.where(valid, inter / ua, _NEG_INF)

    sidx = jax.lax.broadcasted_iota(jnp.int32, (G, 1), 0)
    bto = jnp.max(iou, axis=0, keepdims=True)
    bti = jnp.min(jnp.where(iou == bto, sidx, _IBIG), axis=0,
                  keepdims=True)
    bto_ref[0] = bto
    bti_ref[0] = bti

    m = jnp.max(iou, axis=1, keepdims=True)
    lidx = jax.lax.broadcasted_iota(jnp.int32, (1, iou.shape[1]), 1)
    a = jnp.min(jnp.where(iou == m, lidx, _IBIG), axis=1,
                keepdims=True) + i * _BLKL

    @pl.when(i == 0)
    def _():
        cmax_ref[...] = m
        carg_ref[...] = a

    @pl.when(i > 0)
    def _():
        upd = m > cmax_ref[...]
        cmax_ref[...] = jnp.where(upd, m, cmax_ref[...])
        carg_ref[...] = jnp.where(upd, a, carg_ref[...])

    @pl.when(i == nblk - 1)
    def _():
        bpi_ref[0] = carg_ref[...]


def _make_loss_kernel(A):
    def _loss_kernel(cls_ref, regT_ref, anchT_ref, ann1_ref, ann2_ref,
                     annT1_ref, annT2_ref, bto_ref, bti_ref, bpi_ref,
                     out_ref):
        i = pl.program_id(1)
        C = cls_ref.shape[2]
        ann = ann1_ref[0]
        annn = ann2_ref[0]
        annT = annT1_ref[0]
        annnT = annT2_ref[0]
        bpi = bpi_ref[0]
        G = ann.shape[0]
        blkl = bto_ref.shape[2]
        valid = ann[:, 4:5] != -1.0
        validn = annn[:, 4:5] != -1.0
        sidx = jax.lax.broadcasted_iota(jnp.int32, (G, 1), 0)
        lidx = jax.lax.broadcasted_iota(jnp.int32, (1, blkl), 1)
        gidl = i * blkl + lidx
        in_range = gidl < A

        bto = bto_ref[0]
        bti = bti_ref[0]

        is_best_for = (bpi == gidl) & valid
        chosen = jnp.max(jnp.where(is_best_for, sidx, -1), axis=0,
                         keepdims=True)
        is_best = chosen >= 0
        bto = jnp.where(is_best, 2.0, bto)
        bti = jnp.where(is_best, chosen, bti)

        pos = bto >= 0.5
        neg = bto < 0.4
        active = (pos | neg) & in_range
        npos = jnp.sum(jnp.where(pos, 1.0, 0.0))

        onehot = jnp.where(sidx == bti, 1.0, 0.0)
        asn6 = jax.lax.dot_general(annT, onehot, (((1,), (0,)), ((), ())),
                                   preferred_element_type=jnp.float32)
        labels = asn6[4:5, :].astype(jnp.int32)
        lab_ok = (labels >= 0) & (labels < C)

        eq = (annn[:, 5:6] == asn6[5:6, :]) & validn
        has_match = jnp.max(jnp.where(eq, 1.0, 0.0), axis=0,
                            keepdims=True) == 1.0
        fm = jnp.min(jnp.where(eq, sidx, _IBIG), axis=0, keepdims=True)
        nxt_oh = jnp.where(sidx == fm, 1.0, 0.0)
        ank6 = jax.lax.dot_general(annnT, nxt_oh, (((1,), (0,)), ((), ())),
                                   preferred_element_type=jnp.float32)
        hm_f = jnp.where(has_match, 1.0, 0.0)
        ank = [ank6[k:k + 1, :] * hm_f for k in range(4)]

        labels_s = jnp.transpose(labels)
        c = jnp.clip(cls_ref[0], 1e-4, 1.0 - 1e-4)
        l0 = 0.75 * (c * c) * (-jnp.log(1.0 - c))
        s0 = jnp.sum(l0, axis=1, keepdims=True)
        clsi = jax.lax.broadcasted_iota(jnp.int32, (1, C), 1)
        sel = clsi == labels_s
        c_lab = jnp.sum(jnp.where(sel, c, 0.0), axis=1, keepdims=True)
        s0_l = jnp.transpose(s0)
        c_lab_l = jnp.transpose(c_lab)
        one_m = 1.0 - c_lab_l
        corr = 0.25 * (one_m * one_m) * (-jnp.log(c_lab_l)) \
            - 0.75 * (c_lab_l * c_lab_l) * (-jnp.log(one_m))
        cls_sum = jnp.sum(jnp.where(active, s0_l, 0.0)) \
            + jnp.sum(jnp.where(pos & lab_ok, corr, 0.0))

        at = anchT_ref[...]
        a0, a1, a2, a3 = at[0:1, :], at[1:2, :], at[2:3, :], at[3:4, :]
        aw = a2 - a0
        ah = a3 - a1
        acx = a0 + 0.5 * aw
        acy = a1 + 0.5 * ah
        gw_raw = asn6[2:3, :] - asn6[0:1, :]
        gh_raw = asn6[3:4, :] - asn6[1:2, :]
        gcx = asn6[0:1, :] + 0.5 * gw_raw
        gcy = asn6[1:2, :] + 0.5 * gh_raw
        gwn_raw = ank[2] - ank[0]
        ghn_raw = ank[3] - ank[1]
        gcxn = ank[0] + 0.5 * gwn_raw
        gcyn = ank[1] + 0.5 * ghn_raw
        gw = jnp.maximum(gw_raw, 1.0)
        gh = jnp.maximum(gh_raw, 1.0)
        gwn = jnp.maximum(gwn_raw, 1.0)
        ghn = jnp.maximum(ghn_raw, 1.0)

        t_cols = (
            (gcx - acx) / aw / 0.1,
            (gcy - acy) / ah / 0.1,
            jnp.log(gw / aw) / 0.2,
            jnp.log(gh / ah) / 0.2,
            (gcxn - acx) / aw / 0.1,
            (gcyn - acy) / ah / 0.1,
            jnp.log(gwn / aw) / 0.2,
            jnp.log(ghn / ah) / 0.2,
        )
        regT = jnp.transpose(regT_ref[0])
        reg_sum = 0.0
        for k in range(8):
            rd = jnp.abs(t_cols[k] - regT[k:k + 1, :])
            if k >= 4:
                rd = rd * hm_f
            rl = jnp.where(rd <= 1.0 / 9.0, 0.5 * 9.0 * rd * rd,
                           rd - 0.5 / 9.0)
            reg_sum = reg_sum + jnp.sum(jnp.where(pos, rl, 0.0))

        lane8 = jax.lax.broadcasted_iota(jnp.int32, (1, 8), 1)
        vec = jnp.where(lane8 == 0, cls_sum,
                        jnp.where(lane8 == 1, reg_sum,
                                  jnp.where(lane8 == 2, npos, 0.0)))

        @pl.when(i == 0)
        def _():
            out_ref[0] = vec

        @pl.when(i > 0)
        def _():
            out_ref[0] = out_ref[0] + vec

    return _loss_kernel


def kernel(classifications, regressions, anchors, annotations1, annotations2):
    B, A, C = classifications.shape
    G = annotations1.shape[1]
    a_pad = (-A) % _BLKL
    a_tot = A + a_pad
    nblk = a_tot // _BLKL

    anchT = jnp.pad(anchors[0].astype(jnp.float32).T, ((0, 0), (0, a_pad)),
                    constant_values=-1e30)
    ann1 = annotations1.astype(jnp.float32)
    ann2 = annotations2.astype(jnp.float32)
    annT1 = ann1.transpose(0, 2, 1)
    annT2 = ann2.transpose(0, 2, 1)

    bto_all, bti_all, bpi = pl.pallas_call(
        _assign_kernel,
        grid=(B, nblk),
        in_specs=[
            pl.BlockSpec((4, _BLKL), lambda b, i: (0, i)),
            pl.BlockSpec((1, G, 6), lambda b, i: (b, 0, 0)),
        ],
        out_specs=[
            pl.BlockSpec((1, 1, _BLKL), lambda b, i: (b, 0, i)),
            pl.BlockSpec((1, 1, _BLKL), lambda b, i: (b, 0, i)),
            pl.BlockSpec((1, G, 1), lambda b, i: (b, 0, 0)),
        ],
        out_shape=[
            jax.ShapeDtypeStruct((B, 1, a_tot), jnp.float32),
            jax.ShapeDtypeStruct((B, 1, a_tot), jnp.int32),
            jax.ShapeDtypeStruct((B, G, 1), jnp.int32),
        ],
        scratch_shapes=[
            pltpu.VMEM((G, 1), jnp.float32),
            pltpu.VMEM((G, 1), jnp.int32),
        ],
    )(anchT, ann1)

    sums = pl.pallas_call(
        _make_loss_kernel(A),
        grid=(B, nblk),
        in_specs=[
            pl.BlockSpec((1, _BLKL, C), lambda b, i: (b, i, 0)),
            pl.BlockSpec((1, _BLKL, 8), lambda b, i: (b, i, 0)),
            pl.BlockSpec((4, _BLKL), lambda b, i: (0, i)),
            pl.BlockSpec((1, G, 6), lambda b, i: (b, 0, 0)),
            pl.BlockSpec((1, G, 6), lambda b, i: (b, 0, 0)),
            pl.BlockSpec((1, 6, G), lambda b, i: (b, 0, 0)),
            pl.BlockSpec((1, 6, G), lambda b, i: (b, 0, 0)),
            pl.BlockSpec((1, 1, _BLKL), lambda b, i: (b, 0, i)),
            pl.BlockSpec((1, 1, _BLKL), lambda b, i: (b, 0, i)),
            pl.BlockSpec((1, G, 1), lambda b, i: (b, 0, 0)),
        ],
        out_specs=pl.BlockSpec((1, 1, 8), lambda b, i: (b, 0, 0)),
        out_shape=jax.ShapeDtypeStruct((B, 1, 8), jnp.float32),
    )(classifications, regressions, anchT, ann1, ann2, annT1, annT2,
      bto_all, bti_all, bpi)

    cls_s = sums[:, 0, 0]
    reg_s = sums[:, 0, 1]
    npos = sums[:, 0, 2]
    cls_losses = cls_s / jnp.maximum(npos, 1.0)
    reg_losses = reg_s / jnp.maximum(npos * 8.0, 1.0)
    return (jnp.mean(cls_losses, keepdims=True),
            jnp.mean(reg_losses, keepdims=True))


# speedup vs baseline: 1.1021x; 1.0129x over previous
import jax
import jax.numpy as jnp
from jax.experimental import pallas as pl
from jax.experimental.pallas import tpu as pltpu

_BLKL = 5120
_NEG_INF = float("-inf")
_IBIG = jnp.iinfo(jnp.int32).max


def _assign_kernel(anchT_ref, ann_ref, bto_ref, bti_ref, bpi_ref,
                   cmax_ref, carg_ref):
    i = pl.program_id(1)
    nblk = pl.num_programs(1)
    at = anchT_ref[...]
    ann = ann_ref[0]
    G = ann.shape[0]
    a0, a1, a2, a3 = at[0:1, :], at[1:2, :], at[2:3, :], at[3:4, :]
    b0, b1 = ann[:, 0:1], ann[:, 1:2]
    b2, b3 = ann[:, 2:3], ann[:, 3:4]
    valid = ann[:, 4:5] != -1.0
    area_b = (b2 - b0) * (b3 - b1)
    iw = jnp.maximum(jnp.minimum(a2, b2) - jnp.maximum(a0, b0), 0.0)
    ih = jnp.maximum(jnp.minimum(a3, b3) - jnp.maximum(a1, b1), 0.0)
    inter = iw * ih
    ua = jnp.maximum((a2 - a0) * (a3 - a1) + area_b - inter, 1e-8)
    iou = jnp.where(valid, inter / ua, _NEG_INF)

    sidx = jax.lax.broadcasted_iota(jnp.int32, (G, 1), 0)
    bto = jnp.max(iou, axis=0, keepdims=True)
    bti = jnp.min(jnp.where(iou == bto, sidx, _IBIG), axis=0,
                  keepdims=True)
    bto_ref[0] = bto
    bti_ref[0] = bti

    m = jnp.max(iou, axis=1, keepdims=True)
    lidx = jax.lax.broadcasted_iota(jnp.int32, (1, iou.shape[1]), 1)
    a = jnp.min(jnp.where(iou == m, lidx, _IBIG), axis=1,
                keepdims=True) + i * _BLKL

    @pl.when(i == 0)
    def _():
        cmax_ref[...] = m
        carg_ref[...] = a

    @pl.when(i > 0)
    def _():
        upd = m > cmax_ref[...]
        cmax_ref[...] = jnp.where(upd, m, cmax_ref[...])
        carg_ref[...] = jnp.where(upd, a, carg_ref[...])

    @pl.when(i == nblk - 1)
    def _():
        bpi_ref[0] = carg_ref[...]


def _make_loss_kernel(A):
    def _loss_kernel(cls_ref, regT_ref, anchT_ref, ann1_ref, ann2_ref,
                     annT1_ref, annT2_ref, bto_ref, bti_ref, bpi_ref,
                     out_ref):
        i = pl.program_id(1)
        C = cls_ref.shape[2]
        ann = ann1_ref[0]
        annn = ann2_ref[0]
        annT = annT1_ref[0]
        annnT = annT2_ref[0]
        bpi = bpi_ref[0]
        G = ann.shape[0]
        blkl = bto_ref.shape[2]
        valid = ann[:, 4:5] != -1.0
        validn = annn[:, 4:5] != -1.0
        sidx = jax.lax.broadcasted_iota(jnp.int32, (G, 1), 0)
        lidx = jax.lax.broadcasted_iota(jnp.int32, (1, blkl), 1)
        gidl = i * blkl + lidx
        in_range = gidl < A

        bto = bto_ref[0]
        bti = bti_ref[0]

        is_best_for = (bpi == gidl) & valid
        chosen = jnp.max(jnp.where(is_best_for, sidx, -1), axis=0,
                         keepdims=True)
        is_best = chosen >= 0
        bto = jnp.where(is_best, 2.0, bto)
        bti = jnp.where(is_best, chosen, bti)

        pos = bto >= 0.5
        neg = bto < 0.4
        active = (pos | neg) & in_range
        npos = jnp.sum(jnp.where(pos, 1.0, 0.0))

        onehot = jnp.where(sidx == bti, 1.0, 0.0)
        asn6 = jax.lax.dot_general(annT, onehot, (((1,), (0,)), ((), ())),
                                   preferred_element_type=jnp.float32)
        labels = asn6[4:5, :].astype(jnp.int32)
        lab_ok = (labels >= 0) & (labels < C)

        eq = (annn[:, 5:6] == asn6[5:6, :]) & validn
        has_match = jnp.max(jnp.where(eq, 1.0, 0.0), axis=0,
                            keepdims=True) == 1.0
        fm = jnp.min(jnp.where(eq, sidx, _IBIG), axis=0, keepdims=True)
        nxt_oh = jnp.where(sidx == fm, 1.0, 0.0)
        ank6 = jax.lax.dot_general(annnT, nxt_oh, (((1,), (0,)), ((), ())),
                                   preferred_element_type=jnp.float32)
        hm_f = jnp.where(has_match, 1.0, 0.0)
        ank = [ank6[k:k + 1, :] * hm_f for k in range(4)]

        labels_s = jnp.transpose(labels)
        c = jnp.clip(cls_ref[0], 1e-4, 1.0 - 1e-4)
        l0 = 0.75 * (c * c) * (-jnp.log(1.0 - c))
        s0 = jnp.sum(l0, axis=1, keepdims=True)
        clsi = jax.lax.broadcasted_iota(jnp.int32, (1, C), 1)
        sel = clsi == labels_s
        c_lab = jnp.sum(jnp.where(sel, c, 0.0), axis=1, keepdims=True)
        s0_l = jnp.transpose(s0)
        c_lab_l = jnp.transpose(c_lab)
        one_m = 1.0 - c_lab_l
        corr = 0.25 * (one_m * one_m) * (-jnp.log(c_lab_l)) \
            - 0.75 * (c_lab_l * c_lab_l) * (-jnp.log(one_m))
        cls_sum = jnp.sum(jnp.where(active, s0_l, 0.0)) \
            + jnp.sum(jnp.where(pos & lab_ok, corr, 0.0))

        at = anchT_ref[...]
        a0, a1, a2, a3 = at[0:1, :], at[1:2, :], at[2:3, :], at[3:4, :]
        aw = a2 - a0
        ah = a3 - a1
        acx = a0 + 0.5 * aw
        acy = a1 + 0.5 * ah
        gw_raw = asn6[2:3, :] - asn6[0:1, :]
        gh_raw = asn6[3:4, :] - asn6[1:2, :]
        gcx = asn6[0:1, :] + 0.5 * gw_raw
        gcy = asn6[1:2, :] + 0.5 * gh_raw
        gwn_raw = ank[2] - ank[0]
        ghn_raw = ank[3] - ank[1]
        gcxn = ank[0] + 0.5 * gwn_raw
        gcyn = ank[1] + 0.5 * ghn_raw
        gw = jnp.maximum(gw_raw, 1.0)
        gh = jnp.maximum(gh_raw, 1.0)
        gwn = jnp.maximum(gwn_raw, 1.0)
        ghn = jnp.maximum(ghn_raw, 1.0)

        t_cols = (
            (gcx - acx) / aw / 0.1,
            (gcy - acy) / ah / 0.1,
            jnp.log(gw / aw) / 0.2,
            jnp.log(gh / ah) / 0.2,
            (gcxn - acx) / aw / 0.1,
            (gcyn - acy) / ah / 0.1,
            jnp.log(gwn / aw) / 0.2,
            jnp.log(ghn / ah) / 0.2,
        )
        regT = jnp.transpose(regT_ref[0])
        reg_sum = 0.0
        for k in range(8):
            rd = jnp.abs(t_cols[k] - regT[k:k + 1, :])
            if k >= 4:
                rd = rd * hm_f
            rl = jnp.where(rd <= 1.0 / 9.0, 0.5 * 9.0 * rd * rd,
                           rd - 0.5 / 9.0)
            reg_sum = reg_sum + jnp.sum(jnp.where(pos, rl, 0.0))

        lane8 = jax.lax.broadcasted_iota(jnp.int32, (1, 8), 1)
        vec = jnp.where(lane8 == 0, cls_sum,
                        jnp.where(lane8 == 1, reg_sum,
                                  jnp.where(lane8 == 2, npos, 0.0)))

        @pl.when(i == 0)
        def _():
            out_ref[0] = vec

        @pl.when(i > 0)
        def _():
            out_ref[0] = out_ref[0] + vec

    return _loss_kernel


def kernel(classifications, regressions, anchors, annotations1, annotations2):
    B, A, C = classifications.shape
    G = annotations1.shape[1]
    a_pad = (-A) % _BLKL
    a_tot = A + a_pad
    nblk = a_tot // _BLKL

    anchT = jnp.pad(anchors[0].astype(jnp.float32).T, ((0, 0), (0, a_pad)),
                    constant_values=-1e30)
    ann1 = annotations1.astype(jnp.float32)
    ann2 = annotations2.astype(jnp.float32)
    annT1 = ann1.transpose(0, 2, 1)
    annT2 = ann2.transpose(0, 2, 1)

    bto_all, bti_all, bpi = pl.pallas_call(
        _assign_kernel,
        grid=(B, nblk),
        in_specs=[
            pl.BlockSpec((4, _BLKL), lambda b, i: (0, i)),
            pl.BlockSpec((1, G, 6), lambda b, i: (b, 0, 0)),
        ],
        out_specs=[
            pl.BlockSpec((1, 1, _BLKL), lambda b, i: (b, 0, i)),
            pl.BlockSpec((1, 1, _BLKL), lambda b, i: (b, 0, i)),
            pl.BlockSpec((1, G, 1), lambda b, i: (b, 0, 0)),
        ],
        out_shape=[
            jax.ShapeDtypeStruct((B, 1, a_tot), jnp.float32),
            jax.ShapeDtypeStruct((B, 1, a_tot), jnp.int32),
            jax.ShapeDtypeStruct((B, G, 1), jnp.int32),
        ],
        scratch_shapes=[
            pltpu.VMEM((G, 1), jnp.float32),
            pltpu.VMEM((G, 1), jnp.int32),
        ],
    )(anchT, ann1)

    sums = pl.pallas_call(
        _make_loss_kernel(A),
        grid=(B, nblk),
        in_specs=[
            pl.BlockSpec((1, _BLKL, C), lambda b, i: (b, i, 0)),
            pl.BlockSpec((1, _BLKL, 8), lambda b, i: (b, i, 0)),
            pl.BlockSpec((4, _BLKL), lambda b, i: (0, i)),
            pl.BlockSpec((1, G, 6), lambda b, i: (b, 0, 0)),
            pl.BlockSpec((1, G, 6), lambda b, i: (b, 0, 0)),
            pl.BlockSpec((1, 6, G), lambda b, i: (b, 0, 0)),
            pl.BlockSpec((1, 6, G), lambda b, i: (b, 0, 0)),
            pl.BlockSpec((1, 1, _BLKL), lambda b, i: (b, 0, i)),
            pl.BlockSpec((1, 1, _BLKL), lambda b, i: (b, 0, i)),
            pl.BlockSpec((1, G, 1), lambda b, i: (b, 0, 0)),
        ],
        out_specs=pl.BlockSpec((1, 1, 8), lambda b, i: (b, 0, 0)),
        out_shape=jax.ShapeDtypeStruct((B, 1, 8), jnp.float32),
    )(classifications, regressions, anchT, ann1, ann2, annT1, annT2,
      bto_all, bti_all, bpi)

    cls_s = sums[:, 0, 0]
    reg_s = sums[:, 0, 1]
    npos = sums[:, 0, 2]
    cls_losses = cls_s / jnp.maximum(npos, 1.0)
    reg_losses = reg_s / jnp.maximum(npos * 8.0, 1.0)
    return (jnp.mean(cls_losses, keepdims=True),
            jnp.mean(reg_losses, keepdims=True))


# speedup vs baseline: 1.3720x; 1.2449x over previous
import jax
import jax.numpy as jnp
from jax.experimental import pallas as pl
from jax.experimental.pallas import tpu as pltpu

_BLKL = 5120
_NEG_INF = float("-inf")
_IBIG = jnp.iinfo(jnp.int32).max


def _assign_kernel(anchT_ref, ann_ref, bto_ref, bti_ref, bpi_ref,
                   cmax_ref, carg_ref):
    i = pl.program_id(1)
    nblk = pl.num_programs(1)
    at = anchT_ref[...]
    ann = ann_ref[0]
    G = ann.shape[0]
    a0, a1, a2, a3 = at[0:1, :], at[1:2, :], at[2:3, :], at[3:4, :]
    b0, b1 = ann[:, 0:1], ann[:, 1:2]
    b2, b3 = ann[:, 2:3], ann[:, 3:4]
    valid = ann[:, 4:5] != -1.0
    area_b = (b2 - b0) * (b3 - b1)
    iw = jnp.maximum(jnp.minimum(a2, b2) - jnp.maximum(a0, b0), 0.0)
    ih = jnp.maximum(jnp.minimum(a3, b3) - jnp.maximum(a1, b1), 0.0)
    inter = iw * ih
    ua = jnp.maximum((a2 - a0) * (a3 - a1) + area_b - inter, 1e-8)
    iou = jnp.where(valid, inter / ua, _NEG_INF)

    sidx = jax.lax.broadcasted_iota(jnp.int32, (G, 1), 0)
    bto = jnp.max(iou, axis=0, keepdims=True)
    bti = jnp.min(jnp.where(iou == bto, sidx, _IBIG), axis=0,
                  keepdims=True)
    bto_ref[0] = bto
    bti_ref[0] = bti

    m = jnp.max(iou, axis=1, keepdims=True)
    lidx = jax.lax.broadcasted_iota(jnp.int32, (1, iou.shape[1]), 1)
    a = jnp.min(jnp.where(iou == m, lidx, _IBIG), axis=1,
                keepdims=True) + i * _BLKL

    @pl.when(i == 0)
    def _():
        cmax_ref[...] = m
        carg_ref[...] = a

    @pl.when(i > 0)
    def _():
        upd = m > cmax_ref[...]
        cmax_ref[...] = jnp.where(upd, m, cmax_ref[...])
        carg_ref[...] = jnp.where(upd, a, carg_ref[...])

    @pl.when(i == nblk - 1)
    def _():
        bpi_ref[0] = jnp.where(valid, carg_ref[...], -1)


def _make_loss_kernel(A):
    def _loss_kernel(cls_ref, regT_ref, anchT_ref, ann2_ref,
                     annT1_ref, annT2_ref, bto_ref, bti_ref, bpi_ref,
                     out_ref):
        i = pl.program_id(1)
        C = cls_ref.shape[2]
        annn = ann2_ref[0]
        annT = annT1_ref[0]
        annnT = annT2_ref[0]
        bpi = bpi_ref[0]
        G = annn.shape[0]
        blkl = bto_ref.shape[2]
        validn = annn[:, 4:5] != -1.0
        sidx = jax.lax.broadcasted_iota(jnp.int32, (G, 1), 0)
        lidx = jax.lax.broadcasted_iota(jnp.int32, (1, blkl), 1)
        gidl = i * blkl + lidx
        in_range = gidl < A

        bto = bto_ref[0]
        bti = bti_ref[0]

        is_best_for = bpi == gidl
        chosen = jnp.max(jnp.where(is_best_for, sidx, -1), axis=0,
                         keepdims=True)
        is_best = chosen >= 0
        bto = jnp.where(is_best, 2.0, bto)
        bti = jnp.where(is_best, chosen, bti)

        pos = bto >= 0.5
        neg = bto < 0.4
        active = (pos | neg) & in_range
        npos = jnp.sum(jnp.where(pos, 1.0, 0.0))

        onehot = jnp.where(sidx == bti, 1.0, 0.0)
        asn6 = jax.lax.dot_general(annT, onehot, (((1,), (0,)), ((), ())),
                                   preferred_element_type=jnp.float32)
        labels = asn6[4:5, :].astype(jnp.int32)
        lab_ok = (labels >= 0) & (labels < C)

        eq = (annn[:, 5:6] == asn6[5:6, :]) & validn
        fm = jnp.min(jnp.where(eq, sidx, _IBIG), axis=0, keepdims=True)
        has_match = fm != _IBIG
        nxt_oh = jnp.where(sidx == fm, 1.0, 0.0)
        ank6 = jax.lax.dot_general(annnT, nxt_oh, (((1,), (0,)), ((), ())),
                                   preferred_element_type=jnp.float32)
        hm_f = jnp.where(has_match, 1.0, 0.0)
        ank = [ank6[k:k + 1, :] * hm_f for k in range(4)]

        labels_s = jnp.transpose(labels)
        craw = cls_ref[0]
        c = jnp.where(craw >= 1e-4, craw, 1e-4)
        c = jnp.where(c <= 1.0 - 1e-4, c, 1.0 - 1e-4)
        l0 = 0.75 * (c * c) * (-jnp.log(1.0 - c))
        clsi = jax.lax.broadcasted_iota(jnp.int32, (1, C), 1)
        sel = clsi == labels_s
        activef = jnp.where(active, 1.0, 0.0)
        t0c = jax.lax.dot_general(activef, l0, (((1,), (0,)), ((), ())),
                                  preferred_element_type=jnp.float32)
        ones_c = jnp.ones((1, C), jnp.float32)
        c_lab_l = jax.lax.dot_general(
            ones_c, jnp.where(sel, c, 0.0), (((1,), (1,)), ((), ())),
            preferred_element_type=jnp.float32)
        one_m = 1.0 - c_lab_l
        corr = 0.25 * (one_m * one_m) * (-jnp.log(c_lab_l)) \
            - 0.75 * (c_lab_l * c_lab_l) * (-jnp.log(one_m))
        cls_sum = jnp.sum(t0c) \
            + jnp.sum(jnp.where(pos & lab_ok, corr, 0.0))

        at = anchT_ref[...]
        a0, a1, a2, a3 = at[0:1, :], at[1:2, :], at[2:3, :], at[3:4, :]
        aw = a2 - a0
        ah = a3 - a1
        acx = a0 + 0.5 * aw
        acy = a1 + 0.5 * ah
        gw_raw = asn6[2:3, :] - asn6[0:1, :]
        gh_raw = asn6[3:4, :] - asn6[1:2, :]
        gcx = asn6[0:1, :] + 0.5 * gw_raw
        gcy = asn6[1:2, :] + 0.5 * gh_raw
        gwn_raw = ank[2] - ank[0]
        ghn_raw = ank[3] - ank[1]
        gcxn = ank[0] + 0.5 * gwn_raw
        gcyn = ank[1] + 0.5 * ghn_raw
        gw = jnp.maximum(gw_raw, 1.0)
        gh = jnp.maximum(gh_raw, 1.0)
        gwn = jnp.maximum(gwn_raw, 1.0)
        ghn = jnp.maximum(ghn_raw, 1.0)

        t_cols = (
            (gcx - acx) / aw / 0.1,
            (gcy - acy) / ah / 0.1,
            jnp.log(gw / aw) / 0.2,
            jnp.log(gh / ah) / 0.2,
            (gcxn - acx) / aw / 0.1,
            (gcyn - acy) / ah / 0.1,
            jnp.log(gwn / aw) / 0.2,
            jnp.log(ghn / ah) / 0.2,
        )
        regT = jnp.transpose(regT_ref[0])
        racc = 0.0
        for k in range(8):
            rd = jnp.abs(t_cols[k] - regT[k:k + 1, :])
            if k >= 4:
                rd = rd * hm_f
            rl = jnp.where(rd <= 1.0 / 9.0, 0.5 * 9.0 * rd * rd,
                           rd - 0.5 / 9.0)
            racc = racc + rl
        reg_sum = jnp.sum(jnp.where(pos, racc, 0.0))

        lane8 = jax.lax.broadcasted_iota(jnp.int32, (1, 8), 1)
        vec = jnp.where(lane8 == 0, cls_sum,
                        jnp.where(lane8 == 1, reg_sum,
                                  jnp.where(lane8 == 2, npos, 0.0)))

        @pl.when(i == 0)
        def _():
            out_ref[0] = vec

        @pl.when(i > 0)
        def _():
            out_ref[0] = out_ref[0] + vec

    return _loss_kernel


def kernel(classifications, regressions, anchors, annotations1, annotations2):
    B, A, C = classifications.shape
    G = annotations1.shape[1]
    a_pad = (-A) % _BLKL
    a_tot = A + a_pad
    nblk = a_tot // _BLKL

    anchT = jnp.pad(anchors[0].astype(jnp.float32).T, ((0, 0), (0, a_pad)),
                    constant_values=-1e30)
    ann1 = annotations1.astype(jnp.float32)
    ann2 = annotations2.astype(jnp.float32)
    annT1 = ann1.transpose(0, 2, 1)
    annT2 = ann2.transpose(0, 2, 1)

    bto_all, bti_all, bpi = pl.pallas_call(
        _assign_kernel,
        grid=(B, nblk),
        in_specs=[
            pl.BlockSpec((4, _BLKL), lambda b, i: (0, i)),
            pl.BlockSpec((1, G, 6), lambda b, i: (b, 0, 0)),
        ],
        out_specs=[
            pl.BlockSpec((1, 1, _BLKL), lambda b, i: (b, 0, i)),
            pl.BlockSpec((1, 1, _BLKL), lambda b, i: (b, 0, i)),
            pl.BlockSpec((1, G, 1), lambda b, i: (b, 0, 0)),
        ],
        out_shape=[
            jax.ShapeDtypeStruct((B, 1, a_tot), jnp.float32),
            jax.ShapeDtypeStruct((B, 1, a_tot), jnp.int32),
            jax.ShapeDtypeStruct((B, G, 1), jnp.int32),
        ],
        scratch_shapes=[
            pltpu.VMEM((G, 1), jnp.float32),
            pltpu.VMEM((G, 1), jnp.int32),
        ],
    )(anchT, ann1)

    sums = pl.pallas_call(
        _make_loss_kernel(A),
        grid=(B, nblk),
        in_specs=[
            pl.BlockSpec((1, _BLKL, C), lambda b, i: (b, i, 0)),
            pl.BlockSpec((1, _BLKL, 8), lambda b, i: (b, i, 0)),
            pl.BlockSpec((4, _BLKL), lambda b, i: (0, i)),
            pl.BlockSpec((1, G, 6), lambda b, i: (b, 0, 0)),
            pl.BlockSpec((1, 6, G), lambda b, i: (b, 0, 0)),
            pl.BlockSpec((1, 6, G), lambda b, i: (b, 0, 0)),
            pl.BlockSpec((1, 1, _BLKL), lambda b, i: (b, 0, i)),
            pl.BlockSpec((1, 1, _BLKL), lambda b, i: (b, 0, i)),
            pl.BlockSpec((1, G, 1), lambda b, i: (b, 0, 0)),
        ],
        out_specs=pl.BlockSpec((1, 1, 8), lambda b, i: (b, 0, 0)),
        out_shape=jax.ShapeDtypeStruct((B, 1, 8), jnp.float32),
    )(classifications, regressions, anchT, ann2, annT1, annT2,
      bto_all, bti_all, bpi)

    cls_s = sums[:, 0, 0]
    reg_s = sums[:, 0, 1]
    npos = sums[:, 0, 2]
    cls_losses = cls_s / jnp.maximum(npos, 1.0)
    reg_losses = reg_s / jnp.maximum(npos * 8.0, 1.0)
    return (jnp.mean(cls_losses, keepdims=True),
            jnp.mean(reg_losses, keepdims=True))


# speedup vs baseline: 1.4179x; 1.0334x over previous
import jax
import jax.numpy as jnp
from jax.experimental import pallas as pl
from jax.experimental.pallas import tpu as pltpu

_BLKL = 10240
_NEG_INF = float("-inf")
_IBIG = jnp.iinfo(jnp.int32).max


def _assign_kernel(anchT_ref, ann_ref, bto_ref, bti_ref, bpi_ref,
                   cmax_ref, carg_ref):
    i = pl.program_id(1)
    nblk = pl.num_programs(1)
    at = anchT_ref[...]
    ann = ann_ref[0]
    G = ann.shape[0]
    a0, a1, a2, a3 = at[0:1, :], at[1:2, :], at[2:3, :], at[3:4, :]
    b0, b1 = ann[:, 0:1], ann[:, 1:2]
    b2, b3 = ann[:, 2:3], ann[:, 3:4]
    valid = ann[:, 4:5] != -1.0
    area_b = (b2 - b0) * (b3 - b1)
    iw = jnp.maximum(jnp.minimum(a2, b2) - jnp.maximum(a0, b0), 0.0)
    ih = jnp.maximum(jnp.minimum(a3, b3) - jnp.maximum(a1, b1), 0.0)
    inter = iw * ih
    ua = jnp.maximum((a2 - a0) * (a3 - a1) + area_b - inter, 1e-8)
    iou = jnp.where(valid, inter / ua, _NEG_INF)

    sidx = jax.lax.broadcasted_iota(jnp.int32, (G, 1), 0)
    bto = jnp.max(iou, axis=0, keepdims=True)
    bti = jnp.min(jnp.where(iou == bto, sidx, _IBIG), axis=0,
                  keepdims=True)
    bto_ref[0] = bto
    bti_ref[0] = bti

    m = jnp.max(iou, axis=1, keepdims=True)
    lidx = jax.lax.broadcasted_iota(jnp.int32, (1, iou.shape[1]), 1)
    a = jnp.min(jnp.where(iou == m, lidx, _IBIG), axis=1,
                keepdims=True) + i * _BLKL

    @pl.when(i == 0)
    def _():
        cmax_ref[...] = m
        carg_ref[...] = a

    @pl.when(i > 0)
    def _():
        upd = m > cmax_ref[...]
        cmax_ref[...] = jnp.where(upd, m, cmax_ref[...])
        carg_ref[...] = jnp.where(upd, a, carg_ref[...])

    @pl.when(i == nblk - 1)
    def _():
        bpi_ref[0] = jnp.where(valid, carg_ref[...], -1)


def _make_loss_kernel(A):
    def _loss_kernel(cls_ref, regT_ref, anchT_ref, ann2_ref,
                     annT1_ref, annT2_ref, bto_ref, bti_ref, bpi_ref,
                     out_ref):
        i = pl.program_id(1)
        C = cls_ref.shape[2]
        annn = ann2_ref[0]
        annT = annT1_ref[0]
        annnT = annT2_ref[0]
        bpi = bpi_ref[0]
        G = annn.shape[0]
        blkl = bto_ref.shape[2]
        validn = annn[:, 4:5] != -1.0
        sidx = jax.lax.broadcasted_iota(jnp.int32, (G, 1), 0)
        lidx = jax.lax.broadcasted_iota(jnp.int32, (1, blkl), 1)
        gidl = i * blkl + lidx
        in_range = gidl < A

        bto = bto_ref[0]
        bti = bti_ref[0]

        is_best_for = bpi == gidl
        chosen = jnp.max(jnp.where(is_best_for, sidx, -1), axis=0,
                         keepdims=True)
        is_best = chosen >= 0
        bto = jnp.where(is_best, 2.0, bto)
        bti = jnp.where(is_best, chosen, bti)

        pos = bto >= 0.5
        neg = bto < 0.4
        active = (pos | neg) & in_range
        npos = jnp.sum(jnp.where(pos, 1.0, 0.0))

        onehot = jnp.where(sidx == bti, 1.0, 0.0)
        asn6 = jax.lax.dot_general(annT, onehot, (((1,), (0,)), ((), ())),
                                   preferred_element_type=jnp.float32)
        labels = asn6[4:5, :].astype(jnp.int32)
        lab_ok = (labels >= 0) & (labels < C)

        eq = (annn[:, 5:6] == asn6[5:6, :]) & validn
        fm = jnp.min(jnp.where(eq, sidx, _IBIG), axis=0, keepdims=True)
        has_match = fm != _IBIG
        nxt_oh = jnp.where(sidx == fm, 1.0, 0.0)
        ank6 = jax.lax.dot_general(annnT, nxt_oh, (((1,), (0,)), ((), ())),
                                   preferred_element_type=jnp.float32)
        hm_f = jnp.where(has_match, 1.0, 0.0)
        ank = [ank6[k:k + 1, :] * hm_f for k in range(4)]

        labels_s = jnp.transpose(labels)
        craw = cls_ref[0]
        c = jnp.where(craw >= 1e-4, craw, 1e-4)
        c = jnp.where(c <= 1.0 - 1e-4, c, 1.0 - 1e-4)
        l0 = 0.75 * (c * c) * (-jnp.log(1.0 - c))
        clsi = jax.lax.broadcasted_iota(jnp.int32, (1, C), 1)
        sel = clsi == labels_s
        activef = jnp.where(active, 1.0, 0.0)
        t0c = jax.lax.dot_general(activef, l0, (((1,), (0,)), ((), ())),
                                  preferred_element_type=jnp.float32)
        ones_c = jnp.ones((1, C), jnp.float32)
        c_lab_l = jax.lax.dot_general(
            ones_c, jnp.where(sel, c, 0.0), (((1,), (1,)), ((), ())),
            preferred_element_type=jnp.float32)
        one_m = 1.0 - c_lab_l
        corr = 0.25 * (one_m * one_m) * (-jnp.log(c_lab_l)) \
            - 0.75 * (c_lab_l * c_lab_l) * (-jnp.log(one_m))
        cls_sum = jnp.sum(t0c) \
            + jnp.sum(jnp.where(pos & lab_ok, corr, 0.0))

        at = anchT_ref[...]
        a0, a1, a2, a3 = at[0:1, :], at[1:2, :], at[2:3, :], at[3:4, :]
        aw = a2 - a0
        ah = a3 - a1
        acx = a0 + 0.5 * aw
        acy = a1 + 0.5 * ah
        gw_raw = asn6[2:3, :] - asn6[0:1, :]
        gh_raw = asn6[3:4, :] - asn6[1:2, :]
        gcx = asn6[0:1, :] + 0.5 * gw_raw
        gcy = asn6[1:2, :] + 0.5 * gh_raw
        gwn_raw = ank[2] - ank[0]
        ghn_raw = ank[3] - ank[1]
        gcxn = ank[0] + 0.5 * gwn_raw
        gcyn = ank[1] + 0.5 * ghn_raw
        gw = jnp.maximum(gw_raw, 1.0)
        gh = jnp.maximum(gh_raw, 1.0)
        gwn = jnp.maximum(gwn_raw, 1.0)
        ghn = jnp.maximum(ghn_raw, 1.0)

        t_cols = (
            (gcx - acx) / aw / 0.1,
            (gcy - acy) / ah / 0.1,
            jnp.log(gw / aw) / 0.2,
            jnp.log(gh / ah) / 0.2,
            (gcxn - acx) / aw / 0.1,
            (gcyn - acy) / ah / 0.1,
            jnp.log(gwn / aw) / 0.2,
            jnp.log(ghn / ah) / 0.2,
        )
        regT = jnp.transpose(regT_ref[0])
        racc = 0.0
        for k in range(8):
            rd = jnp.abs(t_cols[k] - regT[k:k + 1, :])
            if k >= 4:
                rd = rd * hm_f
            rl = jnp.where(rd <= 1.0 / 9.0, 0.5 * 9.0 * rd * rd,
                           rd - 0.5 / 9.0)
            racc = racc + rl
        reg_sum = jnp.sum(jnp.where(pos, racc, 0.0))

        lane8 = jax.lax.broadcasted_iota(jnp.int32, (1, 8), 1)
        vec = jnp.where(lane8 == 0, cls_sum,
                        jnp.where(lane8 == 1, reg_sum,
                                  jnp.where(lane8 == 2, npos, 0.0)))

        @pl.when(i == 0)
        def _():
            out_ref[0] = vec

        @pl.when(i > 0)
        def _():
            out_ref[0] = out_ref[0] + vec

    return _loss_kernel


def kernel(classifications, regressions, anchors, annotations1, annotations2):
    B, A, C = classifications.shape
    G = annotations1.shape[1]
    a_pad = (-A) % _BLKL
    a_tot = A + a_pad
    nblk = a_tot // _BLKL

    anchT = jnp.pad(anchors[0].astype(jnp.float32).T, ((0, 0), (0, a_pad)),
                    constant_values=-1e30)
    ann1 = annotations1.astype(jnp.float32)
    ann2 = annotations2.astype(jnp.float32)
    annT1 = ann1.transpose(0, 2, 1)
    annT2 = ann2.transpose(0, 2, 1)

    bto_all, bti_all, bpi = pl.pallas_call(
        _assign_kernel,
        grid=(B, nblk),
        in_specs=[
            pl.BlockSpec((4, _BLKL), lambda b, i: (0, i)),
            pl.BlockSpec((1, G, 6), lambda b, i: (b, 0, 0)),
        ],
        out_specs=[
            pl.BlockSpec((1, 1, _BLKL), lambda b, i: (b, 0, i)),
            pl.BlockSpec((1, 1, _BLKL), lambda b, i: (b, 0, i)),
            pl.BlockSpec((1, G, 1), lambda b, i: (b, 0, 0)),
        ],
        out_shape=[
            jax.ShapeDtypeStruct((B, 1, a_tot), jnp.float32),
            jax.ShapeDtypeStruct((B, 1, a_tot), jnp.int32),
            jax.ShapeDtypeStruct((B, G, 1), jnp.int32),
        ],
        scratch_shapes=[
            pltpu.VMEM((G, 1), jnp.float32),
            pltpu.VMEM((G, 1), jnp.int32),
        ],
    )(anchT, ann1)

    sums = pl.pallas_call(
        _make_loss_kernel(A),
        grid=(B, nblk),
        in_specs=[
            pl.BlockSpec((1, _BLKL, C), lambda b, i: (b, i, 0)),
            pl.BlockSpec((1, _BLKL, 8), lambda b, i: (b, i, 0)),
            pl.BlockSpec((4, _BLKL), lambda b, i: (0, i)),
            pl.BlockSpec((1, G, 6), lambda b, i: (b, 0, 0)),
            pl.BlockSpec((1, 6, G), lambda b, i: (b, 0, 0)),
            pl.BlockSpec((1, 6, G), lambda b, i: (b, 0, 0)),
            pl.BlockSpec((1, 1, _BLKL), lambda b, i: (b, 0, i)),
            pl.BlockSpec((1, 1, _BLKL), lambda b, i: (b, 0, i)),
            pl.BlockSpec((1, G, 1), lambda b, i: (b, 0, 0)),
        ],
        out_specs=pl.BlockSpec((1, 1, 8), lambda b, i: (b, 0, 0)),
        out_shape=jax.ShapeDtypeStruct((B, 1, 8), jnp.float32),
    )(classifications, regressions, anchT, ann2, annT1, annT2,
      bto_all, bti_all, bpi)

    cls_s = sums[:, 0, 0]
    reg_s = sums[:, 0, 1]
    npos = sums[:, 0, 2]
    cls_losses = cls_s / jnp.maximum(npos, 1.0)
    reg_losses = reg_s / jnp.maximum(npos * 8.0, 1.0)
    return (jnp.mean(cls_losses, keepdims=True),
            jnp.mean(reg_losses, keepdims=True))


# speedup vs baseline: 1.4268x; 1.0063x over previous
import jax
import jax.numpy as jnp
from jax.experimental import pallas as pl
from jax.experimental.pallas import tpu as pltpu

_BLKL = 10240
_NEG_INF = float("-inf")
_IBIG = jnp.iinfo(jnp.int32).max


def _assign_kernel(anchT_ref, ann_ref, bto_ref, bti_ref, bpi_ref,
                   cmax_ref, carg_ref):
    i = pl.program_id(1)
    nblk = pl.num_programs(1)
    at = anchT_ref[...]
    ann = ann_ref[0]
    G = ann.shape[0]
    a0, a1, a2, a3 = at[0:1, :], at[1:2, :], at[2:3, :], at[3:4, :]
    b0, b1 = ann[:, 0:1], ann[:, 1:2]
    b2, b3 = ann[:, 2:3], ann[:, 3:4]
    valid = ann[:, 4:5] != -1.0
    area_b = (b2 - b0) * (b3 - b1)
    iw = jnp.maximum(jnp.minimum(a2, b2) - jnp.maximum(a0, b0), 0.0)
    ih = jnp.maximum(jnp.minimum(a3, b3) - jnp.maximum(a1, b1), 0.0)
    inter = iw * ih
    ua = jnp.maximum((a2 - a0) * (a3 - a1) + area_b - inter, 1e-8)
    iou = jnp.where(valid, inter / ua, _NEG_INF)

    sidx = jax.lax.broadcasted_iota(jnp.int32, (G, 1), 0)
    bto = jnp.max(iou, axis=0, keepdims=True)
    bti = jnp.min(jnp.where(iou == bto, sidx, _IBIG), axis=0,
                  keepdims=True)
    bto_ref[0] = bto
    bti_ref[0] = bti

    m = jnp.max(iou, axis=1, keepdims=True)
    lidx = jax.lax.broadcasted_iota(jnp.int32, (1, iou.shape[1]), 1)
    a = jnp.min(jnp.where(iou == m, lidx, _IBIG), axis=1,
                keepdims=True) + i * _BLKL

    @pl.when(i == 0)
    def _():
        cmax_ref[...] = m
        carg_ref[...] = a

    @pl.when(i > 0)
    def _():
        upd = m > cmax_ref[...]
        cmax_ref[...] = jnp.where(upd, m, cmax_ref[...])
        carg_ref[...] = jnp.where(upd, a, carg_ref[...])

    @pl.when(i == nblk - 1)
    def _():
        bpi_ref[0] = jnp.where(valid, carg_ref[...], -1)


def _make_loss_kernel(A):
    def _loss_kernel(cls_ref, regT_ref, anchT_ref, ann2_ref,
                     annT1_ref, annT2_ref, bto_ref, bti_ref, bpi_ref,
                     out_ref):
        i = pl.program_id(1)
        C = cls_ref.shape[2]
        annn = ann2_ref[0]
        annT = annT1_ref[0]
        annnT = annT2_ref[0]
        bpi = bpi_ref[0]
        G = annn.shape[0]
        blkl = bto_ref.shape[2]
        validn = annn[:, 4:5] != -1.0
        sidx = jax.lax.broadcasted_iota(jnp.int32, (G, 1), 0)
        lidx = jax.lax.broadcasted_iota(jnp.int32, (1, blkl), 1)
        gidl = i * blkl + lidx
        in_range = gidl < A

        bto = bto_ref[0]
        bti = bti_ref[0]

        is_best_for = bpi == gidl
        chosen = jnp.max(jnp.where(is_best_for, sidx, -1), axis=0,
                         keepdims=True)
        is_best = chosen >= 0
        bto = jnp.where(is_best, 2.0, bto)
        bti = jnp.where(is_best, chosen, bti)

        pos = bto >= 0.5
        neg = bto < 0.4
        active = (pos | neg) & in_range
        npos = jnp.sum(jnp.where(pos, 1.0, 0.0))

        onehot = jnp.where(sidx == bti, 1.0, 0.0)
        asn6 = jax.lax.dot_general(annT, onehot, (((1,), (0,)), ((), ())),
                                   preferred_element_type=jnp.float32)
        labels = asn6[4:5, :].astype(jnp.int32)
        lab_ok = (labels >= 0) & (labels < C)

        eq = (annn[:, 5:6] == asn6[5:6, :]) & validn
        fm = jnp.min(jnp.where(eq, sidx, _IBIG), axis=0, keepdims=True)
        has_match = fm != _IBIG
        nxt_oh = jnp.where(sidx == fm, 1.0, 0.0)
        ank6 = jax.lax.dot_general(annnT, nxt_oh, (((1,), (0,)), ((), ())),
                                   preferred_element_type=jnp.float32)
        hm_f = jnp.where(has_match, 1.0, 0.0)
        ank = [ank6[k:k + 1, :] * hm_f for k in range(4)]

        labels_s = jnp.transpose(labels)
        craw = jnp.minimum(cls_ref[0], 1.0 - 1e-4)
        c = jnp.where(craw >= 1e-4, craw, 1e-4)
        l0 = 0.75 * (c * c) * (-jnp.log(1.0 - c))
        clsi = jax.lax.broadcasted_iota(jnp.int32, (1, C), 1)
        sel = clsi == labels_s
        activef = jnp.where(active, 1.0, 0.0)
        t0c = jax.lax.dot_general(activef, l0, (((1,), (0,)), ((), ())),
                                  preferred_element_type=jnp.float32)
        ones_c = jnp.ones((1, C), jnp.float32)
        c_lab_l = jax.lax.dot_general(
            ones_c, jnp.where(sel, c, 0.0), (((1,), (1,)), ((), ())),
            preferred_element_type=jnp.float32)
        one_m = 1.0 - c_lab_l
        corr = 0.25 * (one_m * one_m) * (-jnp.log(c_lab_l)) \
            - 0.75 * (c_lab_l * c_lab_l) * (-jnp.log(one_m))
        cls_sum = jnp.sum(t0c) \
            + jnp.sum(jnp.where(pos & lab_ok, corr, 0.0))

        at = anchT_ref[...]
        a0, a1, a2, a3 = at[0:1, :], at[1:2, :], at[2:3, :], at[3:4, :]
        aw = a2 - a0
        ah = a3 - a1
        acx = a0 + 0.5 * aw
        acy = a1 + 0.5 * ah
        gw_raw = asn6[2:3, :] - asn6[0:1, :]
        gh_raw = asn6[3:4, :] - asn6[1:2, :]
        gcx = asn6[0:1, :] + 0.5 * gw_raw
        gcy = asn6[1:2, :] + 0.5 * gh_raw
        gwn_raw = ank[2] - ank[0]
        ghn_raw = ank[3] - ank[1]
        gcxn = ank[0] + 0.5 * gwn_raw
        gcyn = ank[1] + 0.5 * ghn_raw
        gw = jnp.maximum(gw_raw, 1.0)
        gh = jnp.maximum(gh_raw, 1.0)
        gwn = jnp.maximum(gwn_raw, 1.0)
        ghn = jnp.maximum(ghn_raw, 1.0)

        t_cols = (
            (gcx - acx) / aw / 0.1,
            (gcy - acy) / ah / 0.1,
            jnp.log(gw / aw) / 0.2,
            jnp.log(gh / ah) / 0.2,
            (gcxn - acx) / aw / 0.1,
            (gcyn - acy) / ah / 0.1,
            jnp.log(gwn / aw) / 0.2,
            jnp.log(ghn / ah) / 0.2,
        )
        regT = jnp.transpose(regT_ref[0])
        racc = 0.0
        for k in range(8):
            rd = jnp.abs(t_cols[k] - regT[k:k + 1, :])
            if k >= 4:
                rd = rd * hm_f
            rl = jnp.where(rd <= 1.0 / 9.0, 0.5 * 9.0 * rd * rd,
                           rd - 0.5 / 9.0)
            racc = racc + rl
        reg_sum = jnp.sum(jnp.where(pos, racc, 0.0))

        lane8 = jax.lax.broadcasted_iota(jnp.int32, (1, 8), 1)
        vec = jnp.where(lane8 == 0, cls_sum,
                        jnp.where(lane8 == 1, reg_sum,
                                  jnp.where(lane8 == 2, npos, 0.0)))

        @pl.when(i == 0)
        def _():
            out_ref[0] = vec

        @pl.when(i > 0)
        def _():
            out_ref[0] = out_ref[0] + vec

    return _loss_kernel


def kernel(classifications, regressions, anchors, annotations1, annotations2):
    B, A, C = classifications.shape
    G = annotations1.shape[1]
    a_pad = (-A) % _BLKL
    a_tot = A + a_pad
    nblk = a_tot // _BLKL

    anchT = jnp.pad(anchors[0].astype(jnp.float32).T, ((0, 0), (0, a_pad)),
                    constant_values=-1e30)
    ann1 = annotations1.astype(jnp.float32)
    ann2 = annotations2.astype(jnp.float32)
    annT1 = ann1.transpose(0, 2, 1)
    annT2 = ann2.transpose(0, 2, 1)

    bto_all, bti_all, bpi = pl.pallas_call(
        _assign_kernel,
        grid=(B, nblk),
        in_specs=[
            pl.BlockSpec((4, _BLKL), lambda b, i: (0, i)),
            pl.BlockSpec((1, G, 6), lambda b, i: (b, 0, 0)),
        ],
        out_specs=[
            pl.BlockSpec((1, 1, _BLKL), lambda b, i: (b, 0, i)),
            pl.BlockSpec((1, 1, _BLKL), lambda b, i: (b, 0, i)),
            pl.BlockSpec((1, G, 1), lambda b, i: (b, 0, 0)),
        ],
        out_shape=[
            jax.ShapeDtypeStruct((B, 1, a_tot), jnp.float32),
            jax.ShapeDtypeStruct((B, 1, a_tot), jnp.int32),
            jax.ShapeDtypeStruct((B, G, 1), jnp.int32),
        ],
        scratch_shapes=[
            pltpu.VMEM((G, 1), jnp.float32),
            pltpu.VMEM((G, 1), jnp.int32),
        ],
    )(anchT, ann1)

    sums = pl.pallas_call(
        _make_loss_kernel(A),
        grid=(B, nblk),
        in_specs=[
            pl.BlockSpec((1, _BLKL, C), lambda b, i: (b, i, 0)),
            pl.BlockSpec((1, _BLKL, 8), lambda b, i: (b, i, 0)),
            pl.BlockSpec((4, _BLKL), lambda b, i: (0, i)),
            pl.BlockSpec((1, G, 6), lambda b, i: (b, 0, 0)),
            pl.BlockSpec((1, 6, G), lambda b, i: (b, 0, 0)),
            pl.BlockSpec((1, 6, G), lambda b, i: (b, 0, 0)),
            pl.BlockSpec((1, 1, _BLKL), lambda b, i: (b, 0, i)),
            pl.BlockSpec((1, 1, _BLKL), lambda b, i: (b, 0, i)),
            pl.BlockSpec((1, G, 1), lambda b, i: (b, 0, 0)),
        ],
        out_specs=pl.BlockSpec((1, 1, 8), lambda b, i: (b, 0, 0)),
        out_shape=jax.ShapeDtypeStruct((B, 1, 8), jnp.float32),
    )(classifications, regressions, anchT, ann2, annT1, annT2,
      bto_all, bti_all, bpi)

    cls_s = sums[:, 0, 0]
    reg_s = sums[:, 0, 1]
    npos = sums[:, 0, 2]
    cls_losses = cls_s / jnp.maximum(npos, 1.0)
    reg_losses = reg_s / jnp.maximum(npos * 8.0, 1.0)
    return (jnp.mean(cls_losses, keepdims=True),
            jnp.mean(reg_losses, keepdims=True))
